# initial kernel scaffold (unmeasured)
import jax
import jax.numpy as jnp
from jax import lax
from jax.experimental import pallas as pl
from jax.experimental.pallas import tpu as pltpu


def kernel(
    x,
):
    def body(*refs):
        pass

    out_shape = jax.ShapeDtypeStruct(..., jnp.float32)
    return pl.pallas_call(body, out_shape=out_shape)(...)


# baseline (device time: 44405 ns/iter reference)
import jax
import jax.numpy as jnp
from jax import lax
from jax.experimental import pallas as pl
from jax.experimental.pallas import tpu as pltpu

N_DEV = 32
LOG2_N = 5


def kernel(x):
    m, n = x.shape

    def body(x_ref, out_ref, recv_ref, send_sems, recv_sems):
        my_pos = lax.axis_index("i")

        out_ref[...] = x_ref[...]

        for r in range(LOG2_N):
            partner = my_pos ^ (1 << r)
            rdma = pltpu.make_async_remote_copy(
                src_ref=out_ref,
                dst_ref=recv_ref.at[r],
                send_sem=send_sems.at[r],
                recv_sem=recv_sems.at[r],
                device_id=(partner,),
                device_id_type=pl.DeviceIdType.MESH,
            )
            rdma.start()
            rdma.wait()
            out_ref[...] += recv_ref[r]

    return pl.pallas_call(
        body,
        out_shape=jax.ShapeDtypeStruct((m, n), x.dtype),
        in_specs=[pl.BlockSpec(memory_space=pltpu.VMEM)],
        out_specs=pl.BlockSpec(memory_space=pltpu.VMEM),
        scratch_shapes=[
            pltpu.VMEM((LOG2_N, m, n), x.dtype),
            pltpu.SemaphoreType.DMA((LOG2_N,)),
            pltpu.SemaphoreType.DMA((LOG2_N,)),
        ],
    )(x)


# device time: 37022 ns/iter; 1.1994x vs baseline; 1.1994x over previous
import jax
import jax.numpy as jnp
from jax import lax
from jax.experimental import pallas as pl
from jax.experimental.pallas import tpu as pltpu

N_DEV = 32
LOG2_N = 5


def kernel(x):
    m, n = x.shape

    def body(x_ref, out_ref, recv_ref, send_sems, recv_sems):
        my_pos = lax.axis_index("i")

        barrier_sem = pltpu.get_barrier_semaphore()
        for r in range(LOG2_N):
            pl.semaphore_signal(
                barrier_sem, inc=1,
                device_id=(my_pos ^ (1 << r),),
                device_id_type=pl.DeviceIdType.MESH,
            )
        pl.semaphore_wait(barrier_sem, LOG2_N)

        out_ref[...] = x_ref[...]

        for r in range(LOG2_N):
            partner = my_pos ^ (1 << r)
            rdma = pltpu.make_async_remote_copy(
                src_ref=out_ref,
                dst_ref=recv_ref.at[r],
                send_sem=send_sems.at[r],
                recv_sem=recv_sems.at[r],
                device_id=(partner,),
                device_id_type=pl.DeviceIdType.MESH,
            )
            rdma.start()
            rdma.wait()
            out_ref[...] += recv_ref[r]

    return pl.pallas_call(
        body,
        out_shape=jax.ShapeDtypeStruct((m, n), x.dtype),
        in_specs=[pl.BlockSpec(memory_space=pltpu.VMEM)],
        out_specs=pl.BlockSpec(memory_space=pltpu.VMEM),
        scratch_shapes=[
            pltpu.VMEM((LOG2_N, m, n), x.dtype),
            pltpu.SemaphoreType.DMA((LOG2_N,)),
            pltpu.SemaphoreType.DMA((LOG2_N,)),
        ],
        compiler_params=pltpu.CompilerParams(collective_id=0),
    )(x)


# device time: 28656 ns/iter; 1.5496x vs baseline; 1.2919x over previous
import jax
import jax.numpy as jnp
from jax import lax
from jax.experimental import pallas as pl
from jax.experimental.pallas import tpu as pltpu

N_DEV = 32
LOG2_N = 5


def kernel(x):
    m, n = x.shape
    half = m // 2

    def body(x_ref, out_ref, recv_ref, send_sems, recv_sems):
        my_pos = lax.axis_index("i")

        barrier_sem = pltpu.get_barrier_semaphore()
        for b in range(LOG2_N):
            pl.semaphore_signal(
                barrier_sem, inc=1,
                device_id=(my_pos ^ (1 << b),),
                device_id_type=pl.DeviceIdType.MESH,
            )
        pl.semaphore_wait(barrier_sem, LOG2_N)

        out_ref[...] = x_ref[...]

        def make(r, h):
            bit = r if h == 0 else LOG2_N - 1 - r
            return pltpu.make_async_remote_copy(
                src_ref=out_ref.at[pl.ds(h * half, half), :],
                dst_ref=recv_ref.at[r, h],
                send_sem=send_sems.at[r, h],
                recv_sem=recv_sems.at[r, h],
                device_id=(my_pos ^ (1 << bit),),
                device_id_type=pl.DeviceIdType.MESH,
            )

        inflight = {}
        for h in (0, 1):
            inflight[(0, h)] = make(0, h)
            inflight[(0, h)].start()

        for r in range(LOG2_N):
            for h in (0, 1):
                rdma = inflight.pop((r, h))
                rdma.wait()
                out_ref[pl.ds(h * half, half), :] += recv_ref[r, h]
                if r + 1 < LOG2_N:
                    inflight[(r + 1, h)] = make(r + 1, h)
                    inflight[(r + 1, h)].start()

    return pl.pallas_call(
        body,
        out_shape=jax.ShapeDtypeStruct((m, n), x.dtype),
        in_specs=[pl.BlockSpec(memory_space=pltpu.VMEM)],
        out_specs=pl.BlockSpec(memory_space=pltpu.VMEM),
        scratch_shapes=[
            pltpu.VMEM((LOG2_N, 2, half, n), x.dtype),
            pltpu.SemaphoreType.DMA((LOG2_N, 2)),
            pltpu.SemaphoreType.DMA((LOG2_N, 2)),
        ],
        compiler_params=pltpu.CompilerParams(collective_id=0),
    )(x)


# device time: 23940 ns/iter; 1.8548x vs baseline; 1.1970x over previous
import jax
import jax.numpy as jnp
from jax import lax
from jax.experimental import pallas as pl
from jax.experimental.pallas import tpu as pltpu

N_DEV = 32
LOG2_N = 5


N_CHUNK = 4


def kernel(x):
    m, n = x.shape
    rows = m // N_CHUNK

    def body(x_ref, out_ref, recv_ref, send_sems, recv_sems):
        my_pos = lax.axis_index("i")

        barrier_sem = pltpu.get_barrier_semaphore()
        for b in range(LOG2_N):
            pl.semaphore_signal(
                barrier_sem, inc=1,
                device_id=(my_pos ^ (1 << b),),
                device_id_type=pl.DeviceIdType.MESH,
            )
        pl.semaphore_wait(barrier_sem, LOG2_N)

        out_ref[...] = x_ref[...]

        def make(r, c):
            bit = (r + c) % LOG2_N
            return pltpu.make_async_remote_copy(
                src_ref=out_ref.at[pl.ds(c * rows, rows), :],
                dst_ref=recv_ref.at[r, c],
                send_sem=send_sems.at[r, c],
                recv_sem=recv_sems.at[r, c],
                device_id=(my_pos ^ (1 << bit),),
                device_id_type=pl.DeviceIdType.MESH,
            )

        inflight = {}
        for c in range(N_CHUNK):
            inflight[(0, c)] = make(0, c)
            inflight[(0, c)].start()

        for r in range(LOG2_N):
            for c in range(N_CHUNK):
                rdma = inflight.pop((r, c))
                rdma.wait()
                out_ref[pl.ds(c * rows, rows), :] += recv_ref[r, c]
                if r + 1 < LOG2_N:
                    inflight[(r + 1, c)] = make(r + 1, c)
                    inflight[(r + 1, c)].start()

    return pl.pallas_call(
        body,
        out_shape=jax.ShapeDtypeStruct((m, n), x.dtype),
        in_specs=[pl.BlockSpec(memory_space=pltpu.VMEM)],
        out_specs=pl.BlockSpec(memory_space=pltpu.VMEM),
        scratch_shapes=[
            pltpu.VMEM((LOG2_N, N_CHUNK, rows, n), x.dtype),
            pltpu.SemaphoreType.DMA((LOG2_N, N_CHUNK)),
            pltpu.SemaphoreType.DMA((LOG2_N, N_CHUNK)),
        ],
        compiler_params=pltpu.CompilerParams(collective_id=0),
    )(x)


# device time: 22282 ns/iter; 1.9929x vs baseline; 1.0744x over previous
import jax
import jax.numpy as jnp
from jax import lax
from jax.experimental import pallas as pl
from jax.experimental.pallas import tpu as pltpu

N_DEV = 32
LOG2_N = 5
N_CHUNK = 4
MASKS = (1, 3, 4, 8, 16)


def kernel(x):
    m, n = x.shape
    rows = m // N_CHUNK

    def body(x_ref, out_ref, acc_ref, recv_ref, send_sems, recv_sems):
        my_pos = lax.axis_index("i")

        barrier_sem = pltpu.get_barrier_semaphore()
        for mask in MASKS:
            pl.semaphore_signal(
                barrier_sem, inc=1,
                device_id=(my_pos ^ mask,),
                device_id_type=pl.DeviceIdType.MESH,
            )
        pl.semaphore_wait(barrier_sem, LOG2_N)

        def acc_at(r, c):
            ref = x_ref if r == 0 else (out_ref if r == LOG2_N else acc_ref.at[r - 1])
            return ref.at[pl.ds(c * rows, rows), :]

        def make(r, c):
            mask = MASKS[(r + c) % LOG2_N]
            return pltpu.make_async_remote_copy(
                src_ref=acc_at(r, c),
                dst_ref=recv_ref.at[r, c],
                send_sem=send_sems.at[r, c],
                recv_sem=recv_sems.at[r, c],
                device_id=(my_pos ^ mask,),
                device_id_type=pl.DeviceIdType.MESH,
            )

        inflight = {}
        for c in range(N_CHUNK):
            inflight[(0, c)] = make(0, c)
            inflight[(0, c)].start()

        done = []
        for r in range(LOG2_N):
            for c in range(N_CHUNK):
                rdma = inflight.pop((r, c))
                rdma.wait_recv()
                acc_at(r + 1, c)[...] = acc_at(r, c)[...] + recv_ref[r, c]
                if r + 1 < LOG2_N:
                    inflight[(r + 1, c)] = make(r + 1, c)
                    inflight[(r + 1, c)].start()
                done.append(rdma)

        for rdma in done:
            rdma.wait_send()

    return pl.pallas_call(
        body,
        out_shape=jax.ShapeDtypeStruct((m, n), x.dtype),
        in_specs=[pl.BlockSpec(memory_space=pltpu.VMEM)],
        out_specs=pl.BlockSpec(memory_space=pltpu.VMEM),
        scratch_shapes=[
            pltpu.VMEM((LOG2_N - 1, m, n), x.dtype),
            pltpu.VMEM((LOG2_N, N_CHUNK, rows, n), x.dtype),
            pltpu.SemaphoreType.DMA((LOG2_N, N_CHUNK)),
            pltpu.SemaphoreType.DMA((LOG2_N, N_CHUNK)),
        ],
        compiler_params=pltpu.CompilerParams(collective_id=0),
    )(x)
